# Initial kernel scaffold; baseline (speedup 1.0000x reference)
import jax
import jax.numpy as jnp
from jax.experimental import pallas as pl


def kernel(energy, xyz, mol_idx):
    raise NotImplementedError("write your pallas kernel here")


# speedup vs baseline: 40.9040x; 40.9040x over previous
import dataclasses
import functools

import jax
import jax.numpy as jnp
from jax import lax
from jax.experimental import pallas as pl
from jax.experimental.pallas import tpu as pltpu
from jax.experimental.pallas import tpu_sc as plsc

_N_ATOMS = 6400000
_N_MOLS = 50000

_NC = 2
_NS = 16
_NW = _NC * _NS
_LANES = 16
_ATOMS_PER_W = _N_ATOMS // _NW
_WINDOW = 4000
_N_WIN = _ATOMS_PER_W // _WINDOW


def _sc_compiler_params():
    cp = pltpu.CompilerParams()
    if "needs_layout_passes" in pltpu.CompilerParams.__dataclass_fields__:
        cp = dataclasses.replace(cp, needs_layout_passes=False)
    return cp


def _sc_partial_sums(energy, mol_idx):
    mesh = plsc.VectorSubcoreMesh(core_axis_name="c", subcore_axis_name="s")

    @functools.partial(
        pl.kernel,
        mesh=mesh,
        compiler_params=_sc_compiler_params(),
        out_type=jax.ShapeDtypeStruct((_NW, _N_MOLS), jnp.float32),
        scratch_types=[
            pltpu.VMEM((_N_MOLS,), jnp.float32),
            pltpu.VMEM((_WINDOW + _LANES,), jnp.float32),
            pltpu.VMEM((_WINDOW + _LANES,), jnp.float32),
            pltpu.VMEM((_WINDOW + _LANES,), jnp.int32),
            pltpu.VMEM((_WINDOW + _LANES,), jnp.int32),
            pltpu.SemaphoreType.DMA,
            pltpu.SemaphoreType.DMA,
            pltpu.SemaphoreType.DMA,
            pltpu.SemaphoreType.DMA,
        ],
    )
    def sumpool_kernel(energy_hbm, idx_hbm, out_hbm, acc, vbuf0, vbuf1,
                       ibuf0, ibuf1, sv0, sv1, si0, si1):
        wid = lax.axis_index("s") * _NC + lax.axis_index("c")
        base = wid * _ATOMS_PER_W
        sems_v = (sv0, sv1)
        sems_i = (si0, si1)
        vbufs = (vbuf0, vbuf1)
        ibufs = (ibuf0, ibuf1)
        is_last = lax.iota(jnp.int32, _LANES) == (_LANES - 1)

        def start_copy(w, slot):
            off = base + w * _WINDOW
            cv = pltpu.async_copy(
                energy_hbm.at[pl.ds(off, _WINDOW)],
                vbufs[slot].at[pl.ds(0, _WINDOW)], sems_v[slot])
            ci = pltpu.async_copy(
                idx_hbm.at[pl.ds(off, _WINDOW)],
                ibufs[slot].at[pl.ds(0, _WINDOW)], sems_i[slot])
            return cv, ci

        pending = [start_copy(0, 0), None]

        @pl.loop(0, _N_MOLS, step=_LANES)
        def _(o):
            acc[pl.ds(o, _LANES)] = jnp.zeros((_LANES,), jnp.float32)

        pending[1] = start_copy(1, 1)

        for w in range(_N_WIN):
            slot = w % 2
            for h in pending[slot]:
                h.wait()

            ib, vb = ibufs[slot], vbufs[slot]

            @pl.loop(0, _WINDOW, step=_LANES)
            def _(o, ib=ib, vb=vb):
                i = ib[pl.ds(o, _LANES)]
                iup = ib[pl.ds(o + 1, _LANES)]
                v = vb[pl.ds(o, _LANES)]
                c = jnp.cumsum(v)
                d = i != iup
                plsc.addupdate_scatter(acc, [i], c, mask=d | is_last)
                plsc.addupdate_scatter(acc, [iup], -c,
                                       mask=d & jnp.logical_not(is_last))

            if w + 2 < _N_WIN:
                pending[slot] = start_copy(w + 2, slot)

        pltpu.sync_copy(acc, out_hbm.at[wid])

    return sumpool_kernel(energy, mol_idx)


def _tc_combine(partials):
    def body(p_ref, o_ref):
        o_ref[...] = jnp.sum(p_ref[...], axis=0)

    return pl.pallas_call(
        body,
        out_shape=jax.ShapeDtypeStruct((_N_MOLS,), jnp.float32),
    )(partials)


def kernel(energy, xyz, mol_idx):
    del xyz
    partials = _sc_partial_sums(energy, mol_idx.astype(jnp.int32))
    return _tc_combine(partials)


# speedup vs baseline: 92.6391x; 2.2648x over previous
import dataclasses
import functools

import jax
import jax.numpy as jnp
from jax import lax
from jax.experimental import pallas as pl
from jax.experimental.pallas import tpu as pltpu
from jax.experimental.pallas import tpu_sc as plsc

_N_ATOMS = 6400000
_N_MOLS = 50000

_NC = 2
_NS = 16
_NW = _NC * _NS
_LANES = 16
_ATOMS_PER_W = _N_ATOMS // _NW
_WINDOW = 4000
_N_WIN = _ATOMS_PER_W // _WINDOW


def _sc_compiler_params():
    cp = pltpu.CompilerParams()
    if "needs_layout_passes" in pltpu.CompilerParams.__dataclass_fields__:
        cp = dataclasses.replace(cp, needs_layout_passes=False)
    return cp


def _sc_partial_sums(energy, mol_idx):
    mesh = plsc.VectorSubcoreMesh(core_axis_name="c", subcore_axis_name="s")

    @functools.partial(
        pl.kernel,
        mesh=mesh,
        compiler_params=_sc_compiler_params(),
        out_type=jax.ShapeDtypeStruct((_NW, _N_MOLS), jnp.float32),
        scratch_types=[
            pltpu.VMEM((_N_MOLS,), jnp.float32),
            pltpu.VMEM((_WINDOW + _LANES,), jnp.float32),
            pltpu.VMEM((_WINDOW + _LANES,), jnp.float32),
            pltpu.VMEM((_WINDOW + _LANES,), jnp.int32),
            pltpu.VMEM((_WINDOW + _LANES,), jnp.int32),
            pltpu.SemaphoreType.DMA,
            pltpu.SemaphoreType.DMA,
            pltpu.SemaphoreType.DMA,
            pltpu.SemaphoreType.DMA,
        ],
    )
    def sumpool_kernel(energy_hbm, idx_hbm, out_hbm, acc, vbuf0, vbuf1,
                       ibuf0, ibuf1, sv0, sv1, si0, si1):
        wid = lax.axis_index("s") * _NC + lax.axis_index("c")
        base = wid * _ATOMS_PER_W
        sems_v = (sv0, sv1)
        sems_i = (si0, si1)
        vbufs = (vbuf0, vbuf1)
        ibufs = (ibuf0, ibuf1)
        is_last = lax.iota(jnp.int32, _LANES) == (_LANES - 1)

        def start_copy(w, slot):
            off = base + w * _WINDOW
            cv = pltpu.async_copy(
                energy_hbm.at[pl.ds(off, _WINDOW)],
                vbufs[slot].at[pl.ds(0, _WINDOW)], sems_v[slot])
            ci = pltpu.async_copy(
                idx_hbm.at[pl.ds(off, _WINDOW)],
                ibufs[slot].at[pl.ds(0, _WINDOW)], sems_i[slot])
            return cv, ci

        pending = [start_copy(0, 0), None]

        @plsc.parallel_loop(0, _N_MOLS, step=_LANES, unroll=8)
        def _(o):
            acc[pl.ds(o, _LANES)] = jnp.zeros((_LANES,), jnp.float32)

        pending[1] = start_copy(1, 1)

        for w in range(_N_WIN):
            slot = w % 2
            for h in pending[slot]:
                h.wait()

            ib, vb = ibufs[slot], vbufs[slot]

            @plsc.parallel_loop(0, _WINDOW, step=_LANES, unroll=8)
            def _(o, ib=ib, vb=vb):
                i = ib[pl.ds(o, _LANES)]
                iup = ib[pl.ds(o + 1, _LANES)]
                v = vb[pl.ds(o, _LANES)]
                c = jnp.cumsum(v)
                d = i != iup
                plsc.addupdate_scatter(acc, [i], c, mask=d | is_last)
                plsc.addupdate_scatter(acc, [iup], -c,
                                       mask=d & jnp.logical_not(is_last))

            if w + 2 < _N_WIN:
                pending[slot] = start_copy(w + 2, slot)

        pltpu.sync_copy(acc, out_hbm.at[wid])

    return sumpool_kernel(energy, mol_idx)


def _tc_combine(partials):
    def body(p_ref, o_ref):
        o_ref[...] = jnp.sum(p_ref[...], axis=0)

    return pl.pallas_call(
        body,
        out_shape=jax.ShapeDtypeStruct((_N_MOLS,), jnp.float32),
    )(partials)


def kernel(energy, xyz, mol_idx):
    del xyz
    partials = _sc_partial_sums(energy, mol_idx.astype(jnp.int32))
    return _tc_combine(partials)


# speedup vs baseline: 112.6724x; 1.2163x over previous
import dataclasses
import functools

import jax
import jax.numpy as jnp
from jax import lax
from jax.experimental import pallas as pl
from jax.experimental.pallas import tpu as pltpu
from jax.experimental.pallas import tpu_sc as plsc

_N_ATOMS = 6400000
_N_MOLS = 50000

_NC = 2
_NS = 16
_NW = _NC * _NS
_LANES = 16
_ATOMS_PER_W = _N_ATOMS // _NW
_WINDOW = 4000
_N_WIN = _ATOMS_PER_W // _WINDOW
_UNROLL = 10


def _sc_compiler_params():
    cp = pltpu.CompilerParams()
    if "needs_layout_passes" in pltpu.CompilerParams.__dataclass_fields__:
        cp = dataclasses.replace(cp, needs_layout_passes=False)
    return cp


def _sc_partial_sums(energy, mol_idx):
    mesh = plsc.VectorSubcoreMesh(core_axis_name="c", subcore_axis_name="s")

    @functools.partial(
        pl.kernel,
        mesh=mesh,
        compiler_params=_sc_compiler_params(),
        out_type=jax.ShapeDtypeStruct((_NW, _N_MOLS), jnp.float32),
        scratch_types=[
            pltpu.VMEM((_N_MOLS,), jnp.float32),
            pltpu.VMEM((_WINDOW + _LANES,), jnp.float32),
            pltpu.VMEM((_WINDOW + _LANES,), jnp.float32),
            pltpu.VMEM((_WINDOW + _LANES,), jnp.int32),
            pltpu.VMEM((_WINDOW + _LANES,), jnp.int32),
            pltpu.SemaphoreType.DMA,
            pltpu.SemaphoreType.DMA,
            pltpu.SemaphoreType.DMA,
            pltpu.SemaphoreType.DMA,
        ],
    )
    def sumpool_kernel(energy_hbm, idx_hbm, out_hbm, acc, vbuf0, vbuf1,
                       ibuf0, ibuf1, sv0, sv1, si0, si1):
        wid = lax.axis_index("s") * _NC + lax.axis_index("c")
        base = wid * _ATOMS_PER_W
        sems_v = (sv0, sv1)
        sems_i = (si0, si1)
        vbufs = (vbuf0, vbuf1)
        ibufs = (ibuf0, ibuf1)
        lane = lax.iota(jnp.int32, _LANES)
        is_last = lane == (_LANES - 1)
        shift = jnp.minimum(lane + 1, _LANES - 1)

        def start_copy(w, slot):
            off = base + w * _WINDOW
            cv = pltpu.async_copy(
                energy_hbm.at[pl.ds(off, _WINDOW)],
                vbufs[slot].at[pl.ds(0, _WINDOW)], sems_v[slot])
            ci = pltpu.async_copy(
                idx_hbm.at[pl.ds(off, _WINDOW)],
                ibufs[slot].at[pl.ds(0, _WINDOW)], sems_i[slot])
            return cv, ci

        def wait_copy(w, slot):
            off = base + w * _WINDOW
            pltpu.make_async_copy(
                energy_hbm.at[pl.ds(off, _WINDOW)],
                vbufs[slot].at[pl.ds(0, _WINDOW)], sems_v[slot]).wait()
            pltpu.make_async_copy(
                idx_hbm.at[pl.ds(off, _WINDOW)],
                ibufs[slot].at[pl.ds(0, _WINDOW)], sems_i[slot]).wait()

        def process(slot):
            ib, vb = ibufs[slot], vbufs[slot]

            @plsc.parallel_loop(0, _WINDOW, step=_LANES, unroll=_UNROLL)
            def _(o):
                i = ib[pl.ds(o, _LANES)]
                v = vb[pl.ds(o, _LANES)]
                iup = lax.gather(
                    i, shift[:, None],
                    lax.GatherDimensionNumbers(
                        offset_dims=(), collapsed_slice_dims=(0,),
                        start_index_map=(0,)),
                    slice_sizes=(1,),
                    mode=lax.GatherScatterMode.PROMISE_IN_BOUNDS)
                c = jnp.cumsum(v)
                d = i != iup
                plsc.addupdate_scatter(acc, [i], c, mask=d | is_last)
                plsc.addupdate_scatter(acc, [iup], -c, mask=d)

        start_copy(0, 0)

        @plsc.parallel_loop(0, _N_MOLS, step=_LANES, unroll=8)
        def _(o):
            acc[pl.ds(o, _LANES)] = jnp.zeros((_LANES,), jnp.float32)

        start_copy(1, 1)

        @pl.loop(0, _N_WIN, step=2)
        def _(w):
            for p in (0, 1):
                wait_copy(w + p, p)
                process(p)

                @pl.when(w + p + 2 < _N_WIN)
                def _(p=p):
                    start_copy(w + p + 2, p)

        pltpu.sync_copy(acc, out_hbm.at[wid])

    return sumpool_kernel(energy, mol_idx)


def _tc_combine(partials):
    def body(p_ref, o_ref):
        o_ref[...] = jnp.sum(p_ref[...], axis=0)

    return pl.pallas_call(
        body,
        out_shape=jax.ShapeDtypeStruct((_N_MOLS,), jnp.float32),
    )(partials)


def kernel(energy, xyz, mol_idx):
    del xyz
    partials = _sc_partial_sums(energy, mol_idx.astype(jnp.int32))
    return _tc_combine(partials)
